# Initial kernel scaffold; baseline (speedup 1.0000x reference)
import jax
import jax.numpy as jnp
from jax.experimental import pallas as pl


def kernel(x, edge_index, node_time, seed_time, batch_ids, W_feat, b_feat, W_node, b_node, ln_g, ln_b, W_time, b_time, W_self1, W_neigh1, b1, W_self2, W_neigh2, b2, W_head, b_head):
    raise NotImplementedError("write your pallas kernel here")


# speedup vs baseline: 4.5840x; 4.5840x over previous
import functools

import jax
import jax.numpy as jnp
import numpy as np
from jax import lax
from jax.experimental import pallas as pl
from jax.experimental.pallas import tpu as pltpu
from jax.experimental.pallas import tpu_sc as plsc

N = 10000
E = 320000
C = 128
OUT = 128
SEEDS = 1024

NC = 2
NS = 16
NW = NC * NS
L = 16

NPAD = 10240
RELW = NPAD // NW
EW = E // NW
EK = 80
ENCHUNK = EW // EK
ROWS_PER_TILE = N // NS


def _sc_mesh():
    return plsc.VectorSubcoreMesh(core_axis_name="c", subcore_axis_name="s",
                                  num_cores=NC, num_subcores=NS)


def _rel_body(seed_hbm, bid_hbm, nt_hbm, out_hbm, seed_v, bid_v, nt_v, rel_v):
    wid = lax.axis_index("s") * NC + lax.axis_index("c")
    base = wid * RELW
    pltpu.sync_copy(seed_hbm, seed_v)
    pltpu.sync_copy(bid_hbm.at[pl.ds(base, RELW)], bid_v)
    pltpu.sync_copy(nt_hbm.at[pl.ds(base, RELW)], nt_v)
    for i in range(RELW // L):
        idx = bid_v[pl.ds(i * L, L)]
        val = plsc.load_gather(seed_v, [idx])
        rel_v[pl.ds(i * L, L)] = val - nt_v[pl.ds(i * L, L)]
    pltpu.sync_copy(rel_v, out_hbm.at[pl.ds(base, RELW)])


def _sc_rel(seed_time, batch_ids_pad, node_time_pad):
    k = pl.kernel(
        _rel_body,
        out_type=jax.ShapeDtypeStruct((NPAD,), jnp.float32),
        mesh=_sc_mesh(),
        scratch_types=[
            pltpu.VMEM((SEEDS,), jnp.float32),
            pltpu.VMEM((RELW,), jnp.int32),
            pltpu.VMEM((RELW,), jnp.float32),
            pltpu.VMEM((RELW,), jnp.float32),
        ],
        compiler_params=pltpu.CompilerParams(needs_layout_passes=False),
    )
    return k(seed_time, batch_ids_pad, node_time_pad)


ZSTRIPE = NPAD // NS


def _scatter_body(h_hbm, src_hbm, dst_hbm, zeros_hbm, out_hbm,
                  src_v, dst_v, rows_v, agg, sem):
    c = lax.axis_index("c")
    s = lax.axis_index("s")
    wid = s * NC + c

    pltpu.sync_copy(zeros_hbm, agg.at[pl.ds(s * ZSTRIPE, ZSTRIPE)])
    plsc.subcore_barrier()

    e0 = wid * EW

    def chunk(k, _):
        base = e0 + k * EK
        pltpu.sync_copy(src_hbm.at[pl.ds(base, EK)], src_v)
        pltpu.sync_copy(dst_hbm.at[pl.ds(base, EK)], dst_v)
        pltpu.async_copy(h_hbm.at[src_v], rows_v, sem).wait()
        pltpu.sync_copy(rows_v, agg.at[dst_v], add=True)
        return _

    lax.fori_loop(0, ENCHUNK, chunk, None)
    plsc.subcore_barrier()
    wpt = out_hbm.shape[1] // NS
    pltpu.sync_copy(agg.at[pl.ds(s * wpt, wpt)],
                    out_hbm.at[c, pl.ds(s * wpt, wpt)])


def _sc_scatter(h, src, dst, zeros_hbm, out_rows):
    k = pl.kernel(
        _scatter_body,
        out_type=jax.ShapeDtypeStruct((NC, out_rows, C), jnp.float32),
        mesh=_sc_mesh(),
        scratch_types=[
            pltpu.VMEM((EK,), jnp.int32),
            pltpu.VMEM((EK,), jnp.int32),
            pltpu.VMEM((EK, C), jnp.float32),
            pltpu.VMEM_SHARED((NPAD, C), jnp.float32),
            pltpu.SemaphoreType.DMA,
        ],
        compiler_params=pltpu.CompilerParams(needs_layout_passes=False),
    )
    return k(h, src, dst, zeros_hbm)


_RBLK = 2000


def _encoder_body(x_ref, rel_ref, wf_ref, bf_ref, wn_ref, bn_ref,
                  lg_ref, lb_ref, wt_ref, bt_ref, o_ref):
    x = x_ref[...]
    h = jnp.dot(x, wf_ref[...], preferred_element_type=jnp.float32) + bf_ref[...]
    t = jnp.dot(h, wn_ref[...], preferred_element_type=jnp.float32) + bn_ref[...]
    t = jnp.maximum(t, 0.0)
    mu = jnp.mean(t, axis=-1, keepdims=True)
    var = jnp.mean((t - mu) ** 2, axis=-1, keepdims=True)
    t = (t - mu) * lax.rsqrt(var + 1e-5) * lg_ref[...] + lb_ref[...]
    half = C // 2
    f = lax.broadcasted_iota(jnp.int32, (1, half), 1).astype(jnp.float32)
    freqs = jnp.exp(f * (-np.log(10000.0) / half))
    ang = rel_ref[...] * freqs
    pe = jnp.concatenate([jnp.sin(ang), jnp.cos(ang)], axis=-1)
    o_ref[...] = t + jnp.dot(pe, wt_ref[...],
                             preferred_element_type=jnp.float32) + bt_ref[...]


def _tc_encoder(x, rel, W_feat, b_feat, W_node, b_node, ln_g, ln_b,
                W_time, b_time):
    grid = N // _RBLK
    w2 = pl.BlockSpec((C, C), lambda i: (0, 0))
    w1 = pl.BlockSpec((C,), lambda i: (0,))
    return pl.pallas_call(
        _encoder_body,
        grid=(grid,),
        in_specs=[
            pl.BlockSpec((_RBLK, C), lambda i: (i, 0)),
            pl.BlockSpec((_RBLK, 1), lambda i: (i, 0)),
            w2, w1, w2, w1, w1, w1, w2, w1,
        ],
        out_specs=pl.BlockSpec((_RBLK, C), lambda i: (i, 0)),
        out_shape=jax.ShapeDtypeStruct((N, C), jnp.float32),
    )(x, rel, W_feat, b_feat, W_node, b_node, ln_g, ln_b, W_time, b_time)


def _layer_body(h_ref, p0_ref, p1_ref, ws_ref, wn_ref, b_ref, o_ref):
    agg = p0_ref[...] + p1_ref[...]
    o = (jnp.dot(h_ref[...], ws_ref[...], preferred_element_type=jnp.float32)
         + jnp.dot(agg, wn_ref[...], preferred_element_type=jnp.float32)
         + b_ref[...])
    o_ref[...] = jnp.maximum(o, 0.0)


def _tc_layer1(h, p0, p1, W_self, W_neigh, b):
    grid = N // _RBLK
    blk = pl.BlockSpec((_RBLK, C), lambda i: (i, 0))
    w2 = pl.BlockSpec((C, C), lambda i: (0, 0))
    w1 = pl.BlockSpec((C,), lambda i: (0,))
    return pl.pallas_call(
        _layer_body,
        grid=(grid,),
        in_specs=[blk, blk, blk, w2, w2, w1],
        out_specs=blk,
        out_shape=jax.ShapeDtypeStruct((N, C), jnp.float32),
    )(h, p0, p1, W_self, W_neigh, b)


def _head_body(h_ref, q0_ref, q1_ref, ws_ref, wn_ref, b_ref,
               wh_ref, bh_ref, o_ref):
    agg = q0_ref[...] + q1_ref[...]
    t = (jnp.dot(h_ref[...], ws_ref[...], preferred_element_type=jnp.float32)
         + jnp.dot(agg, wn_ref[...], preferred_element_type=jnp.float32)
         + b_ref[...])
    t = jnp.maximum(t, 0.0)
    o_ref[...] = jnp.dot(t, wh_ref[...],
                         preferred_element_type=jnp.float32) + bh_ref[...]


def _tc_head(h_seeds, q0, q1, W_self, W_neigh, b, W_head, b_head):
    return pl.pallas_call(
        _head_body,
        out_shape=jax.ShapeDtypeStruct((SEEDS, OUT), jnp.float32),
    )(h_seeds, q0, q1, W_self, W_neigh, b, W_head, b_head)


def kernel(x, edge_index, node_time, seed_time, batch_ids,
           W_feat, b_feat, W_node, b_node, ln_g, ln_b,
           W_time, b_time,
           W_self1, W_neigh1, b1, W_self2, W_neigh2, b2,
           W_head, b_head):
    pad = NPAD - N
    batch_ids_pad = jnp.concatenate(
        [batch_ids, jnp.zeros((pad,), jnp.int32)])
    node_time_pad = jnp.concatenate(
        [node_time, jnp.zeros((pad,), jnp.float32)])
    rel = _sc_rel(seed_time, batch_ids_pad, node_time_pad)[:N, None]

    h = _tc_encoder(x, rel, W_feat, b_feat, W_node, b_node, ln_g, ln_b,
                    W_time, b_time)

    zeros_hbm = jnp.zeros((ZSTRIPE, C), jnp.float32)
    src = edge_index[0]
    dst = edge_index[1]
    p = _sc_scatter(h, src, dst, zeros_hbm, NPAD)
    h1 = _tc_layer1(h, p[0, :N], p[1, :N], W_self1, W_neigh1, b1)

    q = _sc_scatter(h1, src, dst, zeros_hbm, SEEDS)
    return _tc_head(h1[:SEEDS], q[0], q[1], W_self2, W_neigh2, b2,
                    W_head, b_head)


# speedup vs baseline: 8.0164x; 1.7488x over previous
import functools

import jax
import jax.numpy as jnp
import numpy as np
from jax import lax
from jax.experimental import pallas as pl
from jax.experimental.pallas import tpu as pltpu
from jax.experimental.pallas import tpu_sc as plsc

N = 10000
E = 320000
C = 128
OUT = 128
SEEDS = 1024

NC = 2
NS = 16
NW = NC * NS
L = 16

NPAD = 10240
RELW = NPAD // NW
EW = E // NW
EK = 80
ENCHUNK = EW // EK
ROWS_PER_TILE = N // NS


def _sc_mesh():
    return plsc.VectorSubcoreMesh(core_axis_name="c", subcore_axis_name="s",
                                  num_cores=NC, num_subcores=NS)


def _rel_body(seed_hbm, bid_hbm, nt_hbm, out_hbm, seed_v, bid_v, nt_v, rel_v):
    wid = lax.axis_index("s") * NC + lax.axis_index("c")
    base = wid * RELW
    pltpu.sync_copy(seed_hbm, seed_v)
    pltpu.sync_copy(bid_hbm.at[pl.ds(base, RELW)], bid_v)
    pltpu.sync_copy(nt_hbm.at[pl.ds(base, RELW)], nt_v)
    for i in range(RELW // L):
        idx = bid_v[pl.ds(i * L, L)]
        val = plsc.load_gather(seed_v, [idx])
        rel_v[pl.ds(i * L, L)] = val - nt_v[pl.ds(i * L, L)]
    pltpu.sync_copy(rel_v, out_hbm.at[pl.ds(base, RELW)])


def _sc_rel(seed_time, batch_ids_pad, node_time_pad):
    k = pl.kernel(
        _rel_body,
        out_type=jax.ShapeDtypeStruct((NPAD,), jnp.float32),
        mesh=_sc_mesh(),
        scratch_types=[
            pltpu.VMEM((SEEDS,), jnp.float32),
            pltpu.VMEM((RELW,), jnp.int32),
            pltpu.VMEM((RELW,), jnp.float32),
            pltpu.VMEM((RELW,), jnp.float32),
        ],
        compiler_params=pltpu.CompilerParams(needs_layout_passes=False),
    )
    return k(seed_time, batch_ids_pad, node_time_pad)


ZSTRIPE = NPAD // NS


def _sc_scatter(h, src, dst3d, zeros_hbm, out_rows):
    def body(h_hbm, src_hbm, dst_hbm, zeros_hbm, out_hbm,
             src_v, dst_v, rows0, rows1, agg,
             gsem0, gsem1, ssem0, ssem1):
        rows = [rows0, rows1]
        gsem = [gsem0, gsem1]
        ssem = [ssem0, ssem1]
        c = lax.axis_index("c")
        s = lax.axis_index("s")
        wid = s * NC + c

        pltpu.sync_copy(zeros_hbm, agg.at[pl.ds(s * ZSTRIPE, ZSTRIPE)])
        plsc.subcore_barrier()

        e0 = wid * EW
        pltpu.sync_copy(src_hbm.at[pl.ds(e0, EW)], src_v)
        pltpu.sync_copy(dst_hbm.at[wid], dst_v)

        def fire_gather(k, b):
            pltpu.async_copy(h_hbm.at[src_v.at[pl.ds(k * EK, EK)]],
                             rows[b], gsem[b])

        def wait_gather(b):
            pltpu.make_async_copy(h_hbm.at[pl.ds(0, EK)], rows[b],
                                  gsem[b]).wait()

        def drain_scatter(b):
            pltpu.make_async_copy(h_hbm.at[pl.ds(0, EK)], rows[b],
                                  ssem[b]).wait()

        fire_gather(0, 0)

        def pair(j, _):
            for b in range(2):
                k = 2 * j + b
                o = 1 - b
                wait_gather(b)
                pltpu.async_copy(rows[b], agg.at[dst_v.at[k]], ssem[b],
                                 add=True)

                @pl.when(k >= 1)
                def _():
                    drain_scatter(o)

                fire_gather(k + 1, o)
            return _

        lax.fori_loop(0, ENCHUNK // 2, pair, None)
        kl = ENCHUNK - 1
        wait_gather(kl % 2)
        pltpu.async_copy(rows[kl % 2], agg.at[dst_v.at[kl]],
                         ssem[kl % 2], add=True)
        drain_scatter(1 - (kl % 2))
        drain_scatter(kl % 2)

        plsc.subcore_barrier()
        wpt = out_hbm.shape[1] // NS
        pltpu.sync_copy(agg.at[pl.ds(s * wpt, wpt)],
                        out_hbm.at[c, pl.ds(s * wpt, wpt)])

    k = pl.kernel(
        body,
        out_type=jax.ShapeDtypeStruct((NC, out_rows, C), jnp.float32),
        mesh=_sc_mesh(),
        scratch_types=(
            [pltpu.VMEM((EW,), jnp.int32),
             pltpu.VMEM((ENCHUNK, EK), jnp.int32)]
            + [pltpu.VMEM((EK, C), jnp.float32) for _ in range(2)]
            + [pltpu.VMEM_SHARED((NPAD, C), jnp.float32)]
            + [pltpu.SemaphoreType.DMA for _ in range(4)]
        ),
        compiler_params=pltpu.CompilerParams(needs_layout_passes=False),
    )
    return k(h, src, dst3d, zeros_hbm)


_RBLK = 2000


def _encoder_body(x_ref, rel_ref, wf_ref, bf_ref, wn_ref, bn_ref,
                  lg_ref, lb_ref, wt_ref, bt_ref, o_ref):
    x = x_ref[...]
    h = jnp.dot(x, wf_ref[...], preferred_element_type=jnp.float32) + bf_ref[...]
    t = jnp.dot(h, wn_ref[...], preferred_element_type=jnp.float32) + bn_ref[...]
    t = jnp.maximum(t, 0.0)
    mu = jnp.mean(t, axis=-1, keepdims=True)
    var = jnp.mean((t - mu) ** 2, axis=-1, keepdims=True)
    t = (t - mu) * lax.rsqrt(var + 1e-5) * lg_ref[...] + lb_ref[...]
    half = C // 2
    f = lax.broadcasted_iota(jnp.int32, (1, half), 1).astype(jnp.float32)
    freqs = jnp.exp(f * (-np.log(10000.0) / half))
    ang = rel_ref[...] * freqs
    pe = jnp.concatenate([jnp.sin(ang), jnp.cos(ang)], axis=-1)
    o_ref[...] = t + jnp.dot(pe, wt_ref[...],
                             preferred_element_type=jnp.float32) + bt_ref[...]


def _tc_encoder(x, rel, W_feat, b_feat, W_node, b_node, ln_g, ln_b,
                W_time, b_time):
    grid = N // _RBLK
    w2 = pl.BlockSpec((C, C), lambda i: (0, 0))
    w1 = pl.BlockSpec((C,), lambda i: (0,))
    return pl.pallas_call(
        _encoder_body,
        grid=(grid,),
        in_specs=[
            pl.BlockSpec((_RBLK, C), lambda i: (i, 0)),
            pl.BlockSpec((_RBLK, 1), lambda i: (i, 0)),
            w2, w1, w2, w1, w1, w1, w2, w1,
        ],
        out_specs=pl.BlockSpec((_RBLK, C), lambda i: (i, 0)),
        out_shape=jax.ShapeDtypeStruct((N, C), jnp.float32),
    )(x, rel, W_feat, b_feat, W_node, b_node, ln_g, ln_b, W_time, b_time)


def _layer_body(h_ref, p0_ref, p1_ref, ws_ref, wn_ref, b_ref, o_ref):
    agg = p0_ref[...] + p1_ref[...]
    o = (jnp.dot(h_ref[...], ws_ref[...], preferred_element_type=jnp.float32)
         + jnp.dot(agg, wn_ref[...], preferred_element_type=jnp.float32)
         + b_ref[...])
    o_ref[...] = jnp.maximum(o, 0.0)


def _tc_layer1(h, p0, p1, W_self, W_neigh, b):
    grid = N // _RBLK
    blk = pl.BlockSpec((_RBLK, C), lambda i: (i, 0))
    w2 = pl.BlockSpec((C, C), lambda i: (0, 0))
    w1 = pl.BlockSpec((C,), lambda i: (0,))
    return pl.pallas_call(
        _layer_body,
        grid=(grid,),
        in_specs=[blk, blk, blk, w2, w2, w1],
        out_specs=blk,
        out_shape=jax.ShapeDtypeStruct((N, C), jnp.float32),
    )(h, p0, p1, W_self, W_neigh, b)


def _head_body(h_ref, q0_ref, q1_ref, ws_ref, wn_ref, b_ref,
               wh_ref, bh_ref, o_ref):
    agg = q0_ref[...] + q1_ref[...]
    t = (jnp.dot(h_ref[...], ws_ref[...], preferred_element_type=jnp.float32)
         + jnp.dot(agg, wn_ref[...], preferred_element_type=jnp.float32)
         + b_ref[...])
    t = jnp.maximum(t, 0.0)
    o_ref[...] = jnp.dot(t, wh_ref[...],
                         preferred_element_type=jnp.float32) + bh_ref[...]


def _tc_head(h_seeds, q0, q1, W_self, W_neigh, b, W_head, b_head):
    return pl.pallas_call(
        _head_body,
        out_shape=jax.ShapeDtypeStruct((SEEDS, OUT), jnp.float32),
    )(h_seeds, q0, q1, W_self, W_neigh, b, W_head, b_head)


def kernel(x, edge_index, node_time, seed_time, batch_ids,
           W_feat, b_feat, W_node, b_node, ln_g, ln_b,
           W_time, b_time,
           W_self1, W_neigh1, b1, W_self2, W_neigh2, b2,
           W_head, b_head):
    pad = NPAD - N
    batch_ids_pad = jnp.concatenate(
        [batch_ids, jnp.zeros((pad,), jnp.int32)])
    node_time_pad = jnp.concatenate(
        [node_time, jnp.zeros((pad,), jnp.float32)])
    rel = _sc_rel(seed_time, batch_ids_pad, node_time_pad)[:N, None]

    h = _tc_encoder(x, rel, W_feat, b_feat, W_node, b_node, ln_g, ln_b,
                    W_time, b_time)

    zeros_hbm = jnp.zeros((ZSTRIPE, C), jnp.float32)
    src = edge_index[0]
    dst3d = edge_index[1].reshape(NW, ENCHUNK, EK)
    p = _sc_scatter(h, src, dst3d, zeros_hbm, NPAD)
    h1 = _tc_layer1(h, p[0, :N], p[1, :N], W_self1, W_neigh1, b1)

    q = _sc_scatter(h1, src, dst3d, zeros_hbm, SEEDS)
    return _tc_head(h1[:SEEDS], q[0], q[1], W_self2, W_neigh2, b2,
                    W_head, b_head)


# speedup vs baseline: 9.3135x; 1.1618x over previous
import functools

import jax
import jax.numpy as jnp
import numpy as np
from jax import lax
from jax.experimental import pallas as pl
from jax.experimental.pallas import tpu as pltpu
from jax.experimental.pallas import tpu_sc as plsc

N = 10000
E = 320000
C = 128
OUT = 128
SEEDS = 1024

NC = 2
NS = 16
NW = NC * NS
L = 16

NPAD = 10240
RELW = NPAD // NW
EW = E // NW
EK = 80
ENCHUNK = EW // EK
ROWS_PER_TILE = N // NS


def _sc_mesh():
    return plsc.VectorSubcoreMesh(core_axis_name="c", subcore_axis_name="s",
                                  num_cores=NC, num_subcores=NS)


EWP = EW + EK
CCH = EWP // EK
TRASH = SEEDS
AGG2_ROWS = 1152
Z2STRIPE = AGG2_ROWS // NS


def _prep_body(seed_hbm, bid_hbm, nt_hbm, src_hbm, dst_hbm,
               rel_hbm, csrc_hbm, cdst_hbm, cnt_hbm,
               seed_v, bid_v, nt_v, rel_v, src_in, dst_in,
               csrc_v, cdst_v, cnt_v):
    wid = lax.axis_index("s") * NC + lax.axis_index("c")
    base = wid * RELW
    pltpu.sync_copy(seed_hbm, seed_v)
    pltpu.sync_copy(bid_hbm.at[pl.ds(base, RELW)], bid_v)
    pltpu.sync_copy(nt_hbm.at[pl.ds(base, RELW)], nt_v)
    for i in range(RELW // L):
        idx = bid_v[pl.ds(i * L, L)]
        val = plsc.load_gather(seed_v, [idx])
        rel_v[pl.ds(i * L, L)] = val - nt_v[pl.ds(i * L, L)]
    pltpu.sync_copy(rel_v, rel_hbm.at[pl.ds(base, RELW)])

    e0 = wid * EW
    pltpu.sync_copy(src_hbm.at[pl.ds(e0, EW)], src_in)
    pltpu.sync_copy(dst_hbm.at[pl.ds(e0, EW)], dst_in)

    def step(j, off):
        sv = src_in[pl.ds(j * L, L)]
        dv = dst_in[pl.ds(j * L, L)]
        m = dv < SEEDS
        cum = plsc.cumsum(m.astype(jnp.int32))
        pos = off + cum - 1
        plsc.store_scatter(csrc_v, [pos], sv, mask=m)
        plsc.store_scatter(cdst_v, [pos // EK, pos % EK], dv, mask=m)
        return off + jnp.max(cum)

    off = lax.fori_loop(0, EW // L, step, jnp.int32(0))
    zero_v = jnp.zeros((L,), jnp.int32)
    trash_v = zero_v + TRASH
    ramp = lax.iota(jnp.int32, L)
    for j in range(EK // L):
        pos = off + j * L + ramp
        plsc.store_scatter(csrc_v, [pos], zero_v)
        plsc.store_scatter(cdst_v, [pos // EK, pos % EK], trash_v)
    nch = (off + EK - 1) // EK
    cnt_v[...] = zero_v + nch
    pltpu.sync_copy(csrc_v, csrc_hbm.at[wid])
    pltpu.sync_copy(cdst_v, cdst_hbm.at[wid])
    pltpu.sync_copy(cnt_v, cnt_hbm.at[wid])


def _sc_prep(seed_time, batch_ids_pad, node_time_pad, src, dst):
    k = pl.kernel(
        _prep_body,
        out_type=(
            jax.ShapeDtypeStruct((NPAD,), jnp.float32),
            jax.ShapeDtypeStruct((NW, EWP), jnp.int32),
            jax.ShapeDtypeStruct((NW, CCH, EK), jnp.int32),
            jax.ShapeDtypeStruct((NW, L), jnp.int32),
        ),
        mesh=_sc_mesh(),
        scratch_types=[
            pltpu.VMEM((SEEDS,), jnp.float32),
            pltpu.VMEM((RELW,), jnp.int32),
            pltpu.VMEM((RELW,), jnp.float32),
            pltpu.VMEM((RELW,), jnp.float32),
            pltpu.VMEM((EW,), jnp.int32),
            pltpu.VMEM((EW,), jnp.int32),
            pltpu.VMEM((EWP,), jnp.int32),
            pltpu.VMEM((CCH, EK), jnp.int32),
            pltpu.VMEM((L,), jnp.int32),
        ],
        compiler_params=pltpu.CompilerParams(needs_layout_passes=False),
    )
    return k(seed_time, batch_ids_pad, node_time_pad, src, dst)


ZSTRIPE = NPAD // NS


def _sc_scatter(h, src, dst3d, zeros_hbm, out_rows):
    def body(h_hbm, src_hbm, dst_hbm, zeros_hbm, out_hbm,
             src_v, dst_v, rows0, rows1, agg,
             gsem0, gsem1, ssem0, ssem1):
        rows = [rows0, rows1]
        gsem = [gsem0, gsem1]
        ssem = [ssem0, ssem1]
        c = lax.axis_index("c")
        s = lax.axis_index("s")
        wid = s * NC + c

        pltpu.sync_copy(zeros_hbm, agg.at[pl.ds(s * ZSTRIPE, ZSTRIPE)])
        plsc.subcore_barrier()

        e0 = wid * EW
        pltpu.sync_copy(src_hbm.at[pl.ds(e0, EW)], src_v)
        pltpu.sync_copy(dst_hbm.at[wid], dst_v)

        def fire_gather(k, b):
            pltpu.async_copy(h_hbm.at[src_v.at[pl.ds(k * EK, EK)]],
                             rows[b], gsem[b])

        def wait_gather(b):
            pltpu.make_async_copy(h_hbm.at[pl.ds(0, EK)], rows[b],
                                  gsem[b]).wait()

        def drain_scatter(b):
            pltpu.make_async_copy(h_hbm.at[pl.ds(0, EK)], rows[b],
                                  ssem[b]).wait()

        fire_gather(0, 0)

        def pair(j, _):
            for b in range(2):
                k = 2 * j + b
                o = 1 - b
                wait_gather(b)
                pltpu.async_copy(rows[b], agg.at[dst_v.at[k]], ssem[b],
                                 add=True)

                @pl.when(k >= 1)
                def _():
                    drain_scatter(o)

                fire_gather(k + 1, o)
            return _

        lax.fori_loop(0, ENCHUNK // 2, pair, None)
        kl = ENCHUNK - 1
        wait_gather(kl % 2)
        pltpu.async_copy(rows[kl % 2], agg.at[dst_v.at[kl]],
                         ssem[kl % 2], add=True)
        drain_scatter(1 - (kl % 2))
        drain_scatter(kl % 2)

        plsc.subcore_barrier()
        wpt = out_hbm.shape[1] // NS
        pltpu.sync_copy(agg.at[pl.ds(s * wpt, wpt)],
                        out_hbm.at[c, pl.ds(s * wpt, wpt)])

    k = pl.kernel(
        body,
        out_type=jax.ShapeDtypeStruct((NC, out_rows, C), jnp.float32),
        mesh=_sc_mesh(),
        scratch_types=(
            [pltpu.VMEM((EW,), jnp.int32),
             pltpu.VMEM((ENCHUNK, EK), jnp.int32)]
            + [pltpu.VMEM((EK, C), jnp.float32) for _ in range(2)]
            + [pltpu.VMEM_SHARED((NPAD, C), jnp.float32)]
            + [pltpu.SemaphoreType.DMA for _ in range(4)]
        ),
        compiler_params=pltpu.CompilerParams(needs_layout_passes=False),
    )
    return k(h, src, dst3d, zeros_hbm)


def _sc_scatter_seeds(h1, csrc, cdst, cnt, zeros_hbm):
    def body(h_hbm, csrc_hbm, cdst_hbm, cnt_hbm, zeros_hbm, out_hbm,
             sv, dv, cnt_v, rows0, rows1, agg,
             gsem0, gsem1, ssem0, ssem1):
        rows = [rows0, rows1]
        gsem = [gsem0, gsem1]
        ssem = [ssem0, ssem1]
        c = lax.axis_index("c")
        s = lax.axis_index("s")
        wid = s * NC + c

        pltpu.sync_copy(zeros_hbm.at[pl.ds(0, Z2STRIPE)],
                        agg.at[pl.ds(s * Z2STRIPE, Z2STRIPE)])
        plsc.subcore_barrier()

        pltpu.sync_copy(csrc_hbm.at[wid], sv)
        pltpu.sync_copy(cdst_hbm.at[wid], dv)
        pltpu.sync_copy(cnt_hbm.at[wid], cnt_v)
        nch = jnp.max(cnt_v[...])

        def fire_gather(k, b):
            pltpu.async_copy(h_hbm.at[sv.at[pl.ds(k * EK, EK)]],
                             rows[b], gsem[b])

        def wait_gather(b):
            pltpu.make_async_copy(h_hbm.at[pl.ds(0, EK)], rows[b],
                                  gsem[b]).wait()

        def drain_scatter(b):
            pltpu.make_async_copy(h_hbm.at[pl.ds(0, EK)], rows[b],
                                  ssem[b]).wait()

        @pl.when(nch > 0)
        def _():
            fire_gather(0, 0)

        def pair(j, _):
            for b in range(2):
                k = 2 * j + b

                @pl.when(k < nch)
                def _():
                    wait_gather(b)
                    pltpu.async_copy(rows[b], agg.at[dv.at[k]], ssem[b],
                                     add=True)

                    @pl.when(k >= 1)
                    def _():
                        drain_scatter(1 - b)

                    @pl.when(k + 1 < nch)
                    def _():
                        fire_gather(k + 1, 1 - b)
            return _

        lax.fori_loop(0, (nch + 1) // 2, pair, None)

        @pl.when(nch % 2 == 1)
        def _():
            drain_scatter(0)

        @pl.when((nch > 0) & (nch % 2 == 0))
        def _():
            drain_scatter(1)

        plsc.subcore_barrier()
        wpt = SEEDS // NS
        pltpu.sync_copy(agg.at[pl.ds(s * wpt, wpt)],
                        out_hbm.at[c, pl.ds(s * wpt, wpt)])

    k = pl.kernel(
        body,
        out_type=jax.ShapeDtypeStruct((NC, SEEDS, C), jnp.float32),
        mesh=_sc_mesh(),
        scratch_types=(
            [pltpu.VMEM((EWP,), jnp.int32),
             pltpu.VMEM((CCH, EK), jnp.int32),
             pltpu.VMEM((L,), jnp.int32)]
            + [pltpu.VMEM((EK, C), jnp.float32) for _ in range(2)]
            + [pltpu.VMEM_SHARED((AGG2_ROWS, C), jnp.float32)]
            + [pltpu.SemaphoreType.DMA for _ in range(4)]
        ),
        compiler_params=pltpu.CompilerParams(needs_layout_passes=False),
    )
    return k(h1, csrc, cdst, cnt, zeros_hbm)


_RBLK = 2000


def _encoder_body(x_ref, rel_ref, wf_ref, bf_ref, wn_ref, bn_ref,
                  lg_ref, lb_ref, wt_ref, bt_ref, o_ref):
    x = x_ref[...]
    h = jnp.dot(x, wf_ref[...], preferred_element_type=jnp.float32) + bf_ref[...]
    t = jnp.dot(h, wn_ref[...], preferred_element_type=jnp.float32) + bn_ref[...]
    t = jnp.maximum(t, 0.0)
    mu = jnp.mean(t, axis=-1, keepdims=True)
    var = jnp.mean((t - mu) ** 2, axis=-1, keepdims=True)
    t = (t - mu) * lax.rsqrt(var + 1e-5) * lg_ref[...] + lb_ref[...]
    half = C // 2
    f = lax.broadcasted_iota(jnp.int32, (1, half), 1).astype(jnp.float32)
    freqs = jnp.exp(f * (-np.log(10000.0) / half))
    ang = rel_ref[...] * freqs
    pe = jnp.concatenate([jnp.sin(ang), jnp.cos(ang)], axis=-1)
    o_ref[...] = t + jnp.dot(pe, wt_ref[...],
                             preferred_element_type=jnp.float32) + bt_ref[...]


def _tc_encoder(x, rel_pad, W_feat, b_feat, W_node, b_node, ln_g, ln_b,
                W_time, b_time):
    grid = N // _RBLK
    w2 = pl.BlockSpec((C, C), lambda i: (0, 0))
    w1 = pl.BlockSpec((C,), lambda i: (0,))
    return pl.pallas_call(
        _encoder_body,
        grid=(grid,),
        in_specs=[
            pl.BlockSpec((_RBLK, C), lambda i: (i, 0)),
            pl.BlockSpec((_RBLK, 1), lambda i: (i, 0)),
            w2, w1, w2, w1, w1, w1, w2, w1,
        ],
        out_specs=pl.BlockSpec((_RBLK, C), lambda i: (i, 0)),
        out_shape=jax.ShapeDtypeStruct((N, C), jnp.float32),
    )(x, rel_pad, W_feat, b_feat, W_node, b_node, ln_g, ln_b, W_time, b_time)


def _layer_body(h_ref, p0_ref, p1_ref, ws_ref, wn_ref, b_ref, o_ref):
    agg = p0_ref[0] + p1_ref[0]
    o = (jnp.dot(h_ref[...], ws_ref[...], preferred_element_type=jnp.float32)
         + jnp.dot(agg, wn_ref[...], preferred_element_type=jnp.float32)
         + b_ref[...])
    o_ref[...] = jnp.maximum(o, 0.0)


def _tc_layer1(h, p, W_self, W_neigh, b):
    grid = N // _RBLK
    blk = pl.BlockSpec((_RBLK, C), lambda i: (i, 0))
    p0s = pl.BlockSpec((1, _RBLK, C), lambda i: (0, i, 0))
    p1s = pl.BlockSpec((1, _RBLK, C), lambda i: (1, i, 0))
    w2 = pl.BlockSpec((C, C), lambda i: (0, 0))
    w1 = pl.BlockSpec((C,), lambda i: (0,))
    return pl.pallas_call(
        _layer_body,
        grid=(grid,),
        in_specs=[blk, p0s, p1s, w2, w2, w1],
        out_specs=blk,
        out_shape=jax.ShapeDtypeStruct((N, C), jnp.float32),
    )(h, p, p, W_self, W_neigh, b)


def _head_body(h_ref, q0_ref, q1_ref, ws_ref, wn_ref, b_ref,
               wh_ref, bh_ref, o_ref):
    agg = q0_ref[0] + q1_ref[0]
    t = (jnp.dot(h_ref[...], ws_ref[...], preferred_element_type=jnp.float32)
         + jnp.dot(agg, wn_ref[...], preferred_element_type=jnp.float32)
         + b_ref[...])
    t = jnp.maximum(t, 0.0)
    o_ref[...] = jnp.dot(t, wh_ref[...],
                         preferred_element_type=jnp.float32) + bh_ref[...]


def _tc_head(h1, q, W_self, W_neigh, b, W_head, b_head):
    return pl.pallas_call(
        _head_body,
        grid=(1,),
        in_specs=[
            pl.BlockSpec((SEEDS, C), lambda i: (0, 0)),
            pl.BlockSpec((1, SEEDS, C), lambda i: (0, 0, 0)),
            pl.BlockSpec((1, SEEDS, C), lambda i: (1, 0, 0)),
            pl.BlockSpec((C, C), lambda i: (0, 0)),
            pl.BlockSpec((C, C), lambda i: (0, 0)),
            pl.BlockSpec((C,), lambda i: (0,)),
            pl.BlockSpec((C, OUT), lambda i: (0, 0)),
            pl.BlockSpec((OUT,), lambda i: (0,)),
        ],
        out_specs=pl.BlockSpec((SEEDS, OUT), lambda i: (0, 0)),
        out_shape=jax.ShapeDtypeStruct((SEEDS, OUT), jnp.float32),
    )(h1, q, q, W_self, W_neigh, b, W_head, b_head)


def kernel(x, edge_index, node_time, seed_time, batch_ids,
           W_feat, b_feat, W_node, b_node, ln_g, ln_b,
           W_time, b_time,
           W_self1, W_neigh1, b1, W_self2, W_neigh2, b2,
           W_head, b_head):
    pad = NPAD - N
    batch_ids_pad = jnp.concatenate(
        [batch_ids, jnp.zeros((pad,), jnp.int32)])
    node_time_pad = jnp.concatenate(
        [node_time, jnp.zeros((pad,), jnp.float32)])
    src = edge_index[0]
    dst = edge_index[1]
    rel, csrc, cdst, cnt = _sc_prep(seed_time, batch_ids_pad,
                                    node_time_pad, src, dst)

    h = _tc_encoder(x, rel.reshape(NPAD, 1), W_feat, b_feat,
                    W_node, b_node, ln_g, ln_b, W_time, b_time)

    zeros_hbm = jnp.zeros((ZSTRIPE, C), jnp.float32)
    dst3d = dst.reshape(NW, ENCHUNK, EK)
    p = _sc_scatter(h, src, dst3d, zeros_hbm, NPAD)
    h1 = _tc_layer1(h, p, W_self1, W_neigh1, b1)

    q = _sc_scatter_seeds(h1, csrc, cdst, cnt, zeros_hbm)
    return _tc_head(h1, q, W_self2, W_neigh2, b2, W_head, b_head)


# speedup vs baseline: 11.2321x; 1.2060x over previous
import functools

import jax
import jax.numpy as jnp
import numpy as np
from jax import lax
from jax.experimental import pallas as pl
from jax.experimental.pallas import tpu as pltpu
from jax.experimental.pallas import tpu_sc as plsc

N = 10000
E = 320000
C = 128
OUT = 128
SEEDS = 1024

NC = 2
NS = 16
NW = NC * NS
L = 16

NPAD = 10240
RELW = NPAD // NW
EW = E // NW
EK = 80
ENCHUNK = EW // EK
ROWS_PER_TILE = N // NS


def _sc_mesh():
    return plsc.VectorSubcoreMesh(core_axis_name="c", subcore_axis_name="s",
                                  num_cores=NC, num_subcores=NS)


EWP = EW + EK
CCH = EWP // EK
TRASH = SEEDS
AGG2_ROWS = 1152
Z2STRIPE = AGG2_ROWS // NS


def _prep_body(seed_hbm, bid_hbm, nt_hbm, src_hbm, dst_hbm,
               rel_hbm, csrc_hbm, cdst_hbm, cnt_hbm,
               seed_v, bid_v, nt_v, rel_v, src_in, dst_in,
               csrc_v, cdst_v, cnt_v):
    wid = lax.axis_index("s") * NC + lax.axis_index("c")
    base = wid * RELW
    pltpu.sync_copy(seed_hbm, seed_v)
    pltpu.sync_copy(bid_hbm.at[pl.ds(base, RELW)], bid_v)
    pltpu.sync_copy(nt_hbm.at[pl.ds(base, RELW)], nt_v)
    for i in range(RELW // L):
        idx = bid_v[pl.ds(i * L, L)]
        val = plsc.load_gather(seed_v, [idx])
        rel_v[pl.ds(i * L, L)] = val - nt_v[pl.ds(i * L, L)]
    pltpu.sync_copy(rel_v, rel_hbm.at[pl.ds(base, RELW)])

    e0 = wid * EW
    pltpu.sync_copy(src_hbm.at[pl.ds(e0, EW)], src_in)
    pltpu.sync_copy(dst_hbm.at[pl.ds(e0, EW)], dst_in)

    def step(j, off_v):
        sv = src_in[pl.ds(j * L, L)]
        dv = dst_in[pl.ds(j * L, L)]
        m = dv < SEEDS
        cum = plsc.cumsum(m.astype(jnp.int32))
        pos = off_v + cum - 1
        plsc.store_scatter(csrc_v, [pos], sv, mask=m)
        plsc.store_scatter(cdst_v, [pos // EK, pos % EK], dv, mask=m)
        return off_v + plsc.all_reduce_population_count(m)

    off_v = lax.fori_loop(0, EW // L, step, jnp.zeros((L,), jnp.int32),
                          unroll=2)
    off = jnp.max(off_v)
    zero_v = jnp.zeros((L,), jnp.int32)
    trash_v = zero_v + TRASH
    ramp = lax.iota(jnp.int32, L)
    for j in range(EK // L):
        pos = off + j * L + ramp
        plsc.store_scatter(csrc_v, [pos], zero_v)
        plsc.store_scatter(cdst_v, [pos // EK, pos % EK], trash_v)
    nch = (off + EK - 1) // EK
    cnt_v[...] = zero_v + nch
    pltpu.sync_copy(csrc_v, csrc_hbm.at[wid])
    pltpu.sync_copy(cdst_v, cdst_hbm.at[wid])
    pltpu.sync_copy(cnt_v, cnt_hbm.at[wid])


def _sc_prep(seed_time, batch_ids_pad, node_time_pad, src, dst):
    k = pl.kernel(
        _prep_body,
        out_type=(
            jax.ShapeDtypeStruct((NPAD,), jnp.float32),
            jax.ShapeDtypeStruct((NW, EWP), jnp.int32),
            jax.ShapeDtypeStruct((NW, CCH, EK), jnp.int32),
            jax.ShapeDtypeStruct((NW, L), jnp.int32),
        ),
        mesh=_sc_mesh(),
        scratch_types=[
            pltpu.VMEM((SEEDS,), jnp.float32),
            pltpu.VMEM((RELW,), jnp.int32),
            pltpu.VMEM((RELW,), jnp.float32),
            pltpu.VMEM((RELW,), jnp.float32),
            pltpu.VMEM((EW,), jnp.int32),
            pltpu.VMEM((EW,), jnp.int32),
            pltpu.VMEM((EWP,), jnp.int32),
            pltpu.VMEM((CCH, EK), jnp.int32),
            pltpu.VMEM((L,), jnp.int32),
        ],
        compiler_params=pltpu.CompilerParams(needs_layout_passes=False),
    )
    return k(seed_time, batch_ids_pad, node_time_pad, src, dst)


ZSTRIPE = NPAD // NS


def _sc_scatter(h, src, dst3d, zeros_hbm):
    def body(h_hbm, src_hbm, dst_hbm, zeros_hbm, out_hbm,
             sidx0, sidx1, sidx2, dst_v, rows0, rows1, rows2, agg,
             gsem0, gsem1, gsem2, ssem0, ssem1, ssem2,
             isem0, isem1, isem2):
        sidx = [sidx0, sidx1, sidx2]
        isem = [isem0, isem1, isem2]
        rows = [rows0, rows1, rows2]
        gsem = [gsem0, gsem1, gsem2]
        ssem = [ssem0, ssem1, ssem2]
        c = lax.axis_index("c")
        s = lax.axis_index("s")
        wid = s * NC + c

        @pl.when(s < NS - 1)
        def _():
            pltpu.sync_copy(zeros_hbm, agg.at[pl.ds(s * ZSTRIPE, ZSTRIPE)])

        @pl.when(s == NS - 1)
        def _():
            pltpu.sync_copy(zeros_hbm.at[pl.ds(0, N - (NS - 1) * ZSTRIPE)],
                            agg.at[pl.ds((NS - 1) * ZSTRIPE,
                                         N - (NS - 1) * ZSTRIPE)])

        plsc.subcore_barrier()

        e0 = wid * EW
        pltpu.sync_copy(dst_hbm.at[wid], dst_v)
        for kp in range(3):
            pltpu.sync_copy(src_hbm.at[pl.ds(e0 + kp * EK, EK)], sidx[kp])

        def fire_idx(k, b):
            pltpu.async_copy(src_hbm.at[pl.ds(e0 + k * EK, EK)],
                             sidx[b], isem[b])

        def wait_idx(b):
            pltpu.make_async_copy(src_hbm.at[pl.ds(0, EK)], sidx[b],
                                  isem[b]).wait()

        def fire_gather(k, b):
            pltpu.async_copy(h_hbm.at[sidx[b]], rows[b], gsem[b])

        def wait_gather(b):
            pltpu.make_async_copy(h_hbm.at[pl.ds(0, EK)], rows[b],
                                  gsem[b]).wait()

        def drain_scatter(b):
            pltpu.make_async_copy(h_hbm.at[pl.ds(0, EK)], rows[b],
                                  ssem[b]).wait()

        fire_gather(0, 0)
        fire_gather(1, 1)

        def triple(j, _):
            for b in range(3):
                k = 3 * j + b
                wait_gather(b)
                pltpu.async_copy(rows[b], agg.at[dst_v.at[k]], ssem[b],
                                 add=True)

                @pl.when(k + 3 < ENCHUNK)
                def _():
                    fire_idx(k + 3, b)

                @pl.when(k >= 1)
                def _():
                    drain_scatter((b + 2) % 3)

                @pl.when(k >= 1)
                def _():
                    wait_idx((b + 2) % 3)

                fire_gather(k + 2, (b + 2) % 3)
            return _

        nmain = (ENCHUNK - 2) // 3
        lax.fori_loop(0, nmain, triple, None)
        for k in range(3 * nmain, ENCHUNK):
            b = k % 3
            wait_gather(b)
            pltpu.async_copy(rows[b], agg.at[dst_v.at[k]], ssem[b],
                             add=True)
            drain_scatter((b + 2) % 3)
        drain_scatter((ENCHUNK - 1) % 3)

        plsc.subcore_barrier()
        @pl.when(s < NS - 1)
        def _():
            pltpu.sync_copy(agg.at[pl.ds(s * ZSTRIPE, ZSTRIPE)],
                            out_hbm.at[c, pl.ds(s * ZSTRIPE, ZSTRIPE)])

        @pl.when(s == NS - 1)
        def _():
            tail = N - (NS - 1) * ZSTRIPE
            pltpu.sync_copy(agg.at[pl.ds((NS - 1) * ZSTRIPE, tail)],
                            out_hbm.at[c, pl.ds((NS - 1) * ZSTRIPE, tail)])

    k = pl.kernel(
        body,
        out_type=jax.ShapeDtypeStruct((NC, N, C), jnp.float32),
        mesh=_sc_mesh(),
        scratch_types=(
            [pltpu.VMEM((EK,), jnp.int32) for _ in range(3)]
            + [pltpu.VMEM((ENCHUNK, EK), jnp.int32)]
            + [pltpu.VMEM((EK, C), jnp.float32) for _ in range(3)]
            + [pltpu.VMEM_SHARED((N, C), jnp.float32)]
            + [pltpu.SemaphoreType.DMA for _ in range(9)]
        ),
        compiler_params=pltpu.CompilerParams(needs_layout_passes=False),
    )
    return k(h, src, dst3d, zeros_hbm)


def _sc_scatter_seeds(h1, csrc, cdst, cnt, zeros_hbm):
    def body(h_hbm, csrc_hbm, cdst_hbm, cnt_hbm, zeros_hbm, out_hbm,
             sv, dv, cnt_v, rows0, rows1, rows2, rows3, agg,
             gsem0, gsem1, gsem2, gsem3, ssem0, ssem1, ssem2, ssem3):
        rows = [rows0, rows1, rows2, rows3]
        gsem = [gsem0, gsem1, gsem2, gsem3]
        ssem = [ssem0, ssem1, ssem2, ssem3]
        c = lax.axis_index("c")
        s = lax.axis_index("s")
        wid = s * NC + c

        pltpu.sync_copy(zeros_hbm.at[pl.ds(0, Z2STRIPE)],
                        agg.at[pl.ds(s * Z2STRIPE, Z2STRIPE)])
        plsc.subcore_barrier()

        pltpu.sync_copy(csrc_hbm.at[wid], sv)
        pltpu.sync_copy(cdst_hbm.at[wid], dv)
        pltpu.sync_copy(cnt_hbm.at[wid], cnt_v)
        nch = jnp.max(cnt_v[...])

        def fire_gather(k, b):
            pltpu.async_copy(h_hbm.at[sv.at[pl.ds(k * EK, EK)]],
                             rows[b], gsem[b])

        def wait_gather(b):
            pltpu.make_async_copy(h_hbm.at[pl.ds(0, EK)], rows[b],
                                  gsem[b]).wait()

        def drain_scatter(b):
            pltpu.make_async_copy(h_hbm.at[pl.ds(0, EK)], rows[b],
                                  ssem[b]).wait()

        for kp in range(2):
            @pl.when(kp < nch)
            def _():
                fire_gather(kp, kp)

        def quad(j, _):
            for b in range(4):
                k = 4 * j + b

                @pl.when(k < nch)
                def _():
                    wait_gather(b)
                    pltpu.async_copy(rows[b], agg.at[dv.at[k]], ssem[b],
                                     add=True)

                    @pl.when(k >= 2)
                    def _():
                        drain_scatter((b + 2) % 4)

                    @pl.when(k + 2 < nch)
                    def _():
                        fire_gather(k + 2, (b + 2) % 4)
            return _

        lax.fori_loop(0, (nch + 3) // 4, quad, None)
        for d in range(4):
            @pl.when((nch >= 1) & ((nch - 1) % 4 == d))
            def _():
                drain_scatter(d)
        for d in range(4):
            @pl.when((nch >= 2) & ((nch - 2) % 4 == d))
            def _():
                drain_scatter(d)

        plsc.subcore_barrier()
        wpt = SEEDS // NS
        pltpu.sync_copy(agg.at[pl.ds(s * wpt, wpt)],
                        out_hbm.at[c, pl.ds(s * wpt, wpt)])

    k = pl.kernel(
        body,
        out_type=jax.ShapeDtypeStruct((NC, SEEDS, C), jnp.float32),
        mesh=_sc_mesh(),
        scratch_types=(
            [pltpu.VMEM((EWP,), jnp.int32),
             pltpu.VMEM((CCH, EK), jnp.int32),
             pltpu.VMEM((L,), jnp.int32)]
            + [pltpu.VMEM((EK, C), jnp.float32) for _ in range(4)]
            + [pltpu.VMEM_SHARED((AGG2_ROWS, C), jnp.float32)]
            + [pltpu.SemaphoreType.DMA for _ in range(8)]
        ),
        compiler_params=pltpu.CompilerParams(needs_layout_passes=False),
    )
    return k(h1, csrc, cdst, cnt, zeros_hbm)


_RBLK = 2000


def _encoder_body(x_ref, rel_ref, wf_ref, bf_ref, wn_ref, bn_ref,
                  lg_ref, lb_ref, wt_ref, bt_ref, o_ref):
    x = x_ref[...]
    h = jnp.dot(x, wf_ref[...], preferred_element_type=jnp.float32) + bf_ref[...]
    t = jnp.dot(h, wn_ref[...], preferred_element_type=jnp.float32) + bn_ref[...]
    t = jnp.maximum(t, 0.0)
    mu = jnp.mean(t, axis=-1, keepdims=True)
    var = jnp.mean((t - mu) ** 2, axis=-1, keepdims=True)
    t = (t - mu) * lax.rsqrt(var + 1e-5) * lg_ref[...] + lb_ref[...]
    half = C // 2
    f = lax.broadcasted_iota(jnp.int32, (1, half), 1).astype(jnp.float32)
    freqs = jnp.exp(f * (-np.log(10000.0) / half))
    ang = rel_ref[...] * freqs
    pe = jnp.concatenate([jnp.sin(ang), jnp.cos(ang)], axis=-1)
    o_ref[...] = t + jnp.dot(pe, wt_ref[...],
                             preferred_element_type=jnp.float32) + bt_ref[...]


def _tc_encoder(x, rel_pad, W_feat, b_feat, W_node, b_node, ln_g, ln_b,
                W_time, b_time):
    grid = N // _RBLK
    w2 = pl.BlockSpec((C, C), lambda i: (0, 0))
    w1 = pl.BlockSpec((C,), lambda i: (0,))
    return pl.pallas_call(
        _encoder_body,
        grid=(grid,),
        in_specs=[
            pl.BlockSpec((_RBLK, C), lambda i: (i, 0)),
            pl.BlockSpec((_RBLK, 1), lambda i: (i, 0)),
            w2, w1, w2, w1, w1, w1, w2, w1,
        ],
        out_specs=pl.BlockSpec((_RBLK, C), lambda i: (i, 0)),
        out_shape=jax.ShapeDtypeStruct((N, C), jnp.float32),
    )(x, rel_pad, W_feat, b_feat, W_node, b_node, ln_g, ln_b, W_time, b_time)


def _layer_body(h_ref, p0_ref, p1_ref, ws_ref, wn_ref, b_ref, o_ref):
    agg = p0_ref[0] + p1_ref[0]
    o = (jnp.dot(h_ref[...], ws_ref[...], preferred_element_type=jnp.float32)
         + jnp.dot(agg, wn_ref[...], preferred_element_type=jnp.float32)
         + b_ref[...])
    o_ref[...] = jnp.maximum(o, 0.0)


def _tc_layer1(h, p, W_self, W_neigh, b):
    grid = N // _RBLK
    blk = pl.BlockSpec((_RBLK, C), lambda i: (i, 0))
    p0s = pl.BlockSpec((1, _RBLK, C), lambda i: (0, i, 0))
    p1s = pl.BlockSpec((1, _RBLK, C), lambda i: (1, i, 0))
    w2 = pl.BlockSpec((C, C), lambda i: (0, 0))
    w1 = pl.BlockSpec((C,), lambda i: (0,))
    return pl.pallas_call(
        _layer_body,
        grid=(grid,),
        in_specs=[blk, p0s, p1s, w2, w2, w1],
        out_specs=blk,
        out_shape=jax.ShapeDtypeStruct((N, C), jnp.float32),
    )(h, p, p, W_self, W_neigh, b)


def _head_body(h_ref, q0_ref, q1_ref, ws_ref, wn_ref, b_ref,
               wh_ref, bh_ref, o_ref):
    agg = q0_ref[0] + q1_ref[0]
    t = (jnp.dot(h_ref[...], ws_ref[...], preferred_element_type=jnp.float32)
         + jnp.dot(agg, wn_ref[...], preferred_element_type=jnp.float32)
         + b_ref[...])
    t = jnp.maximum(t, 0.0)
    o_ref[...] = jnp.dot(t, wh_ref[...],
                         preferred_element_type=jnp.float32) + bh_ref[...]


def _tc_head(h1, q, W_self, W_neigh, b, W_head, b_head):
    return pl.pallas_call(
        _head_body,
        grid=(1,),
        in_specs=[
            pl.BlockSpec((SEEDS, C), lambda i: (0, 0)),
            pl.BlockSpec((1, SEEDS, C), lambda i: (0, 0, 0)),
            pl.BlockSpec((1, SEEDS, C), lambda i: (1, 0, 0)),
            pl.BlockSpec((C, C), lambda i: (0, 0)),
            pl.BlockSpec((C, C), lambda i: (0, 0)),
            pl.BlockSpec((C,), lambda i: (0,)),
            pl.BlockSpec((C, OUT), lambda i: (0, 0)),
            pl.BlockSpec((OUT,), lambda i: (0,)),
        ],
        out_specs=pl.BlockSpec((SEEDS, OUT), lambda i: (0, 0)),
        out_shape=jax.ShapeDtypeStruct((SEEDS, OUT), jnp.float32),
    )(h1, q, q, W_self, W_neigh, b, W_head, b_head)


def kernel(x, edge_index, node_time, seed_time, batch_ids,
           W_feat, b_feat, W_node, b_node, ln_g, ln_b,
           W_time, b_time,
           W_self1, W_neigh1, b1, W_self2, W_neigh2, b2,
           W_head, b_head):
    pad = NPAD - N
    batch_ids_pad = jnp.concatenate(
        [batch_ids, jnp.zeros((pad,), jnp.int32)])
    node_time_pad = jnp.concatenate(
        [node_time, jnp.zeros((pad,), jnp.float32)])
    src = edge_index[0]
    dst = edge_index[1]
    rel, csrc, cdst, cnt = _sc_prep(seed_time, batch_ids_pad,
                                    node_time_pad, src, dst)

    h = _tc_encoder(x, rel.reshape(NPAD, 1), W_feat, b_feat,
                    W_node, b_node, ln_g, ln_b, W_time, b_time)

    zeros_hbm = jnp.zeros((ZSTRIPE, C), jnp.float32)
    dst3d = dst.reshape(NW, ENCHUNK, EK)
    p = _sc_scatter(h, src, dst3d, zeros_hbm)
    h1 = _tc_layer1(h, p, W_self1, W_neigh1, b1)

    q = _sc_scatter_seeds(h1, csrc, cdst, cnt, zeros_hbm)
    return _tc_head(h1, q, W_self2, W_neigh2, b2, W_head, b_head)
